# Initial kernel scaffold; baseline (speedup 1.0000x reference)
import jax
import jax.numpy as jnp
from jax.experimental import pallas as pl


def kernel(h, edge_index, W1, b1, W2, b2):
    raise NotImplementedError("write your pallas kernel here")


# speedup vs baseline: 4.1383x; 4.1383x over previous
import jax
import jax.numpy as jnp
from jax import lax
from jax.experimental import pallas as pl
from jax.experimental.pallas import tpu as pltpu
from jax.experimental.pallas import tpu_sc as plsc

N = 10000
D = 128
E = 320000

NC = 2
NS = 16
NPAD = 10240
HALF = NPAD // NC
AROWS = HALF + NS
E_PER = E // NS
K = 80
NCHUNK = E_PER // K
ROWS_PER_TILE = HALF // NS
ZROWS = 64


def _seg_body(x_hbm, src_hbm, dst_hbm, out_hbm,
              src_v, dst_v, rows, zbuf, agg, sem):
    cid = lax.axis_index("c")
    sid = lax.axis_index("s")

    z = jnp.zeros((16,), jnp.float32)

    def zrow(r, carry):
        for j in range(D // 16):
            zbuf[r, pl.ds(j * 16, 16)] = z
        return carry

    lax.fori_loop(0, ZROWS, zrow, 0)
    row0 = sid * ROWS_PER_TILE
    for t in range(ROWS_PER_TILE // ZROWS):
        pltpu.sync_copy(zbuf, agg.at[pl.ds(row0 + t * ZROWS, ZROWS)])
    plsc.subcore_barrier()

    pltpu.sync_copy(src_hbm.at[sid], src_v)
    pltpu.sync_copy(dst_hbm.at[sid], dst_v)

    base = cid * HALF
    trash = jnp.full((16,), HALF, jnp.int32) + sid

    def rebase(r, carry):
        for j in range(K // 16):
            v = dst_v[r, pl.ds(j * 16, 16)] - base
            ok = (v >= 0) & (v < HALF)
            dst_v[r, pl.ds(j * 16, 16)] = jnp.where(ok, v, trash)
        return carry

    lax.fori_loop(0, NCHUNK, rebase, 0)

    def chunk(ci, carry):
        pltpu.async_copy(x_hbm.at[src_v.at[ci]], rows, sem).wait()
        pltpu.sync_copy(rows, agg.at[dst_v.at[ci]], add=True)
        return carry

    lax.fori_loop(0, NCHUNK, chunk, 0)
    plsc.subcore_barrier()

    pltpu.sync_copy(agg.at[pl.ds(row0, ROWS_PER_TILE)],
                    out_hbm.at[pl.ds(base + row0, ROWS_PER_TILE)])


@jax.jit
def _seg_sum(x, src_r, dst_r):
    mesh = plsc.VectorSubcoreMesh(core_axis_name="c", subcore_axis_name="s")
    return pl.kernel(
        _seg_body,
        out_type=jax.ShapeDtypeStruct((NPAD, D), jnp.float32),
        mesh=mesh,
        scratch_types=[
            pltpu.VMEM((NCHUNK, K), jnp.int32),
            pltpu.VMEM((NCHUNK, K), jnp.int32),
            pltpu.VMEM((K, D), jnp.float32),
            pltpu.VMEM((ZROWS, D), jnp.float32),
            pltpu.VMEM_SHARED((AROWS, D), jnp.float32),
            pltpu.SemaphoreType.DMA,
        ],
    )(x, src_r, dst_r)


def _mlp_body(p_ref, w_ref, b_ref, o_ref):
    y = jnp.dot(p_ref[...], w_ref[...],
                preferred_element_type=jnp.float32) + b_ref[...]
    o_ref[...] = jnp.maximum(y, 0.0)


@jax.jit
def _mlp(p, W, b):
    R = 1024
    return pl.pallas_call(
        _mlp_body,
        grid=(NPAD // R,),
        in_specs=[
            pl.BlockSpec((R, D), lambda i: (i, 0)),
            pl.BlockSpec((D, D), lambda i: (0, 0)),
            pl.BlockSpec((1, D), lambda i: (0, 0)),
        ],
        out_specs=pl.BlockSpec((R, D), lambda i: (i, 0)),
        out_shape=jax.ShapeDtypeStruct((NPAD, D), jnp.float32),
    )(p, W, b.reshape(1, D))


def kernel(h, edge_index, W1, b1, W2, b2):
    src_r = edge_index[0].reshape(NS, NCHUNK, K)
    dst_r = edge_index[1].reshape(NS, NCHUNK, K)
    a1 = _seg_sum(h, src_r, dst_r)
    x = _mlp(a1, W1, b1)
    a2 = _seg_sum(x, src_r, dst_r)
    return _mlp(a2, W2, b2)[:N]


# speedup vs baseline: 5.4397x; 1.3145x over previous
import jax
import jax.numpy as jnp
from jax import lax
from jax.experimental import pallas as pl
from jax.experimental.pallas import tpu as pltpu
from jax.experimental.pallas import tpu_sc as plsc

N = 10000
D = 128
E = 320000

NC = 2
NS = 16
NPAD = 10240
HALF = NPAD // NC
AROWS = HALF + NS
E_PER = E // NS
K = 80
NCHUNK = E_PER // K
ROWS_PER_TILE = HALF // NS
ZROWS = 16


def _seg_body(x_hbm, src_hbm, dst_hbm, out_hbm,
              src_v, dst_v, rows0, rows1, zbuf, agg, sem):
    cid = lax.axis_index("c")
    sid = lax.axis_index("s")

    z = jnp.zeros((16,), jnp.float32)

    def zrow(r, carry):
        for j in range(D // 16):
            zbuf[r, pl.ds(j * 16, 16)] = z
        return carry

    lax.fori_loop(0, ZROWS, zrow, 0)
    row0 = sid * ROWS_PER_TILE
    for t in range(ROWS_PER_TILE // ZROWS):
        pltpu.sync_copy(zbuf, agg.at[pl.ds(row0 + t * ZROWS, ZROWS)])
    plsc.subcore_barrier()

    pltpu.sync_copy(src_hbm.at[sid], src_v)
    pltpu.sync_copy(dst_hbm.at[sid], dst_v)

    base = cid * HALF
    trash = jnp.full((16,), HALF, jnp.int32) + sid

    def rebase(r, carry):
        for j in range(K // 16):
            v = dst_v[r, pl.ds(j * 16, 16)] - base
            ok = (v >= 0) & (v < HALF)
            dst_v[r, pl.ds(j * 16, 16)] = jnp.where(ok, v, trash)
        return carry

    lax.fori_loop(0, NCHUNK, rebase, 0)

    pltpu.async_copy(x_hbm.at[src_v.at[0]], rows0, sem)

    def chunk2(c0, carry):
        pltpu.make_async_copy(x_hbm.at[src_v.at[c0]], rows0, sem).wait()
        pltpu.async_copy(x_hbm.at[src_v.at[c0 + 1]], rows1, sem)
        pltpu.sync_copy(rows0, agg.at[dst_v.at[c0]], add=True)
        pltpu.make_async_copy(x_hbm.at[src_v.at[c0 + 1]], rows1, sem).wait()

        @pl.when(c0 + 2 < NCHUNK)
        def _():
            pltpu.async_copy(x_hbm.at[src_v.at[c0 + 2]], rows0, sem)

        pltpu.sync_copy(rows1, agg.at[dst_v.at[c0 + 1]], add=True)
        return carry

    lax.fori_loop(0, NCHUNK // 2, lambda i, c: chunk2(i * 2, c), 0)
    plsc.subcore_barrier()

    pltpu.sync_copy(agg.at[pl.ds(row0, ROWS_PER_TILE)],
                    out_hbm.at[pl.ds(base + row0, ROWS_PER_TILE)])


@jax.jit
def _seg_sum(x, src_r, dst_r):
    mesh = plsc.VectorSubcoreMesh(core_axis_name="c", subcore_axis_name="s")
    return pl.kernel(
        _seg_body,
        out_type=jax.ShapeDtypeStruct((NPAD, D), jnp.float32),
        mesh=mesh,
        scratch_types=[
            pltpu.VMEM((NCHUNK, K), jnp.int32),
            pltpu.VMEM((NCHUNK, K), jnp.int32),
            pltpu.VMEM((K, D), jnp.float32),
            pltpu.VMEM((K, D), jnp.float32),
            pltpu.VMEM((ZROWS, D), jnp.float32),
            pltpu.VMEM_SHARED((AROWS, D), jnp.float32),
            pltpu.SemaphoreType.DMA,
        ],
    )(x, src_r, dst_r)


def _mlp_body(p_ref, w_ref, b_ref, o_ref):
    y = jnp.dot(p_ref[...], w_ref[...],
                preferred_element_type=jnp.float32) + b_ref[...]
    o_ref[...] = jnp.maximum(y, 0.0)


@jax.jit
def _mlp(p, W, b):
    R = 1024
    return pl.pallas_call(
        _mlp_body,
        grid=(NPAD // R,),
        in_specs=[
            pl.BlockSpec((R, D), lambda i: (i, 0)),
            pl.BlockSpec((D, D), lambda i: (0, 0)),
            pl.BlockSpec((1, D), lambda i: (0, 0)),
        ],
        out_specs=pl.BlockSpec((R, D), lambda i: (i, 0)),
        out_shape=jax.ShapeDtypeStruct((NPAD, D), jnp.float32),
    )(p, W, b.reshape(1, D))


def kernel(h, edge_index, W1, b1, W2, b2):
    src_r = edge_index[0].reshape(NS, NCHUNK, K)
    dst_r = edge_index[1].reshape(NS, NCHUNK, K)
    a1 = _seg_sum(h, src_r, dst_r)
    x = _mlp(a1, W1, b1)
    a2 = _seg_sum(x, src_r, dst_r)
    return _mlp(a2, W2, b2)[:N]
